# baseline (speedup 1.0000x reference)
import functools

import numpy as np
import jax
import jax.numpy as jnp
from jax import lax
from jax.experimental import pallas as pl
from jax.experimental.pallas import tpu as pltpu
from jax.experimental.pallas import tpu_sc as plsc

F32 = jnp.float32
_HI = lax.Precision.HIGHEST

DIMC = 128
AC = 16
HC = 8
TC_ = 2
NSPC = 64
RBC = 16
CUTC = 5.0

_cov_np = np.linspace(0.3, 1.6, NSPC).astype(np.float32)

NB = 1000
EB = 3200
EBG = 2560
CH = 80

DST_D = 384
VIE_D = 128
EPREP_D = 32


def _swish(x):
    return x / (1.0 + jnp.exp(-x))


def _lnorm(x):
    mu = jnp.mean(x, axis=-1, keepdims=True)
    dx = x - mu
    var = jnp.mean(dx * dx, axis=-1, keepdims=True)
    return dx * lax.rsqrt(1e-06 + var)


def _dot(a, b):
    return jnp.dot(a, b, precision=_HI, preferred_element_type=F32)


def _sc_env():
    info = plsc.get_sparse_core_info()
    return info.num_cores, info.num_subcores


def _node_init(sp2, Zt, Wsl, Wq, Wk, Wv, Wt):
    N = sp2.shape[0]

    def body(sp_ref, zt_ref, wsl_ref, wq_ref, wk_ref, wv_ref, wt_ref,
             xi_ref, src_ref, dst_ref):
        sp = sp_ref[...]
        ids = lax.broadcasted_iota(jnp.int32, (NB, NSPC), 1)
        oh = (ids == sp).astype(F32)
        z = _dot(oh, zt_ref[...])
        xi = _lnorm(_dot(z, wsl_ref[...]))
        q = _dot(xi, wq_ref[...])
        k = _dot(xi, wk_ref[...])
        v = _dot(xi, wv_ref[...])
        t = _dot(xi, wt_ref[...])
        xi_ref[...] = xi
        src_ref[...] = q
        dst_ref[...] = jnp.concatenate(
            [k, v, t, jnp.zeros((NB, DST_D - 258), F32)], axis=1)

    wspec = lambda r, c: pl.BlockSpec((r, c), lambda i: (0, 0))
    return pl.pallas_call(
        body,
        grid=(N // NB,),
        in_specs=[
            pl.BlockSpec((NB, 1), lambda i: (i, 0)),
            wspec(NSPC, 16), wspec(16, DIMC),
            wspec(DIMC, DIMC), wspec(DIMC, DIMC), wspec(DIMC, DIMC),
            wspec(DIMC, TC_),
        ],
        out_specs=[
            pl.BlockSpec((NB, DIMC), lambda i: (i, 0)),
            pl.BlockSpec((NB, DIMC), lambda i: (i, 0)),
            pl.BlockSpec((NB, DST_D), lambda i: (i, 0)),
        ],
        out_shape=[
            jax.ShapeDtypeStruct((N, DIMC), F32),
            jax.ShapeDtypeStruct((N, DIMC), F32),
            jax.ShapeDtypeStruct((N, DST_D), F32),
        ],
    )(sp2, Zt, Wsl, Wq, Wk, Wv, Wt)


def _sc_rcij(species, cov1d, esrc, edst):
    N = species.shape[0]
    E = esrc.shape[0]
    NC, NS = _sc_env()
    e_per_w = E // (NC * NS)
    G = e_per_w // CH
    mesh = plsc.VectorSubcoreMesh(core_axis_name="c", subcore_axis_name="s")

    @functools.partial(
        pl.kernel,
        mesh=mesh,
        out_type=jax.ShapeDtypeStruct((E,), F32),
        scratch_types=[
            pltpu.VMEM((N,), jnp.int32),
            pltpu.VMEM((NSPC,), F32),
            pltpu.VMEM((CH,), jnp.int32),
            pltpu.VMEM((CH,), jnp.int32),
            pltpu.VMEM((CH,), F32),
        ],
        compiler_params=pltpu.CompilerParams(needs_layout_passes=False),
    )
    def rk(sp_hbm, cov_hbm, es_hbm, ed_hbm, out_hbm, sp_v, cov_v, is_v, id_v,
           o_v):
        wid = lax.axis_index("s") * NC + lax.axis_index("c")
        base = wid * e_per_w
        pltpu.sync_copy(sp_hbm, sp_v)
        pltpu.sync_copy(cov_hbm, cov_v)

        def step(g, carry):
            off = base + g * CH
            pltpu.sync_copy(es_hbm.at[pl.ds(off, CH)], is_v)
            pltpu.sync_copy(ed_hbm.at[pl.ds(off, CH)], id_v)
            for j in range(CH // 16):
                si = is_v[pl.ds(16 * j, 16)]
                di = id_v[pl.ds(16 * j, 16)]
                sps = plsc.load_gather(sp_v, [si])
                spd = plsc.load_gather(sp_v, [di])
                vs = plsc.load_gather(cov_v, [sps])
                vd = plsc.load_gather(cov_v, [spd])
                o_v[pl.ds(16 * j, 16)] = vs + vd
            pltpu.sync_copy(o_v, out_hbm.at[pl.ds(off, CH)])
            return carry

        lax.fori_loop(0, G, step, 0)

    return rk(species, cov1d, esrc, edst)


def _sc_gather2(tbl_s, tbl_d, esrc, edst, ch=CH):
    CH = ch
    N, DS = tbl_s.shape
    DD = tbl_d.shape[1]
    E = esrc.shape[0]
    NC, NS = _sc_env()
    e_per_w = E // (NC * NS)
    G = e_per_w // CH
    mesh = plsc.VectorSubcoreMesh(core_axis_name="c", subcore_axis_name="s")

    @functools.partial(
        pl.kernel,
        mesh=mesh,
        out_type=[
            jax.ShapeDtypeStruct((E, DS), F32),
            jax.ShapeDtypeStruct((E, DD), F32),
        ],
        scratch_types=[
            pltpu.VMEM((CH,), jnp.int32),
            pltpu.VMEM((CH,), jnp.int32),
            pltpu.VMEM((CH,), jnp.int32),
            pltpu.VMEM((CH,), jnp.int32),
            pltpu.VMEM((CH, DS), F32),
            pltpu.VMEM((CH, DD), F32),
            pltpu.VMEM((CH, DS), F32),
            pltpu.VMEM((CH, DD), F32),
            pltpu.SemaphoreType.DMA,
            pltpu.SemaphoreType.DMA,
            pltpu.SemaphoreType.DMA,
            pltpu.SemaphoreType.DMA,
        ],
        compiler_params=pltpu.CompilerParams(use_tc_tiling_on_sc=True),
    )
    def gk(ts_hbm, td_hbm, es_hbm, ed_hbm, os_hbm, od_hbm,
           is0, id0, is1, id1, rs0, rd0, rs1, rd1, sg0, sg1, so0, so1):
        wid = lax.axis_index("s") * NC + lax.axis_index("c")
        base = wid * e_per_w
        T = G // 2

        def fetch(off, is_v, id_v, rs_v, rd_v, sg):
            pltpu.sync_copy(es_hbm.at[pl.ds(off, CH)], is_v)
            pltpu.sync_copy(ed_hbm.at[pl.ds(off, CH)], id_v)
            pltpu.async_copy(ts_hbm.at[is_v], rs_v, sg)
            pltpu.async_copy(td_hbm.at[id_v], rd_v, sg)

        def wait_gather(is_v, id_v, rs_v, rd_v, sg):
            pltpu.make_async_copy(ts_hbm.at[is_v], rs_v, sg).wait()
            pltpu.make_async_copy(td_hbm.at[id_v], rd_v, sg).wait()

        def put(off, rs_v, rd_v, so):
            pltpu.async_copy(rs_v, os_hbm.at[pl.ds(off, CH)], so)
            pltpu.async_copy(rd_v, od_hbm.at[pl.ds(off, CH)], so)

        def wait_put(rs_v, rd_v, so):
            pltpu.make_async_copy(rs_v, os_hbm.at[pl.ds(0, CH)], so).wait()
            pltpu.make_async_copy(rd_v, od_hbm.at[pl.ds(0, CH)], so).wait()

        def body(t, carry):
            off0 = base + (2 * t) * CH
            off1 = off0 + CH

            @pl.when(t > 0)
            def _():
                wait_put(rs0, rd0, so0)

            fetch(off0, is0, id0, rs0, rd0, sg0)

            @pl.when(t > 0)
            def _():
                wait_put(rs1, rd1, so1)

            fetch(off1, is1, id1, rs1, rd1, sg1)
            wait_gather(is0, id0, rs0, rd0, sg0)
            put(off0, rs0, rd0, so0)
            wait_gather(is1, id1, rs1, rd1, sg1)
            put(off1, rs1, rd1, so1)
            return carry

        lax.fori_loop(0, T, body, 0)
        if G % 2:
            offt = base + (G - 1) * CH
            wait_put(rs0, rd0, so0)
            fetch(offt, is0, id0, rs0, rd0, sg0)
            wait_gather(is0, id0, rs0, rd0, sg0)
            put(offt, rs0, rd0, so0)
        wait_put(rs0, rd0, so0)
        wait_put(rs1, rd1, so1)

    return gk(tbl_s, tbl_d, esrc, edst)


def _sc_scatter_add2(rows_a, rows_b, idx, N, ch=CH):
    CH = ch
    E, D = rows_a.shape
    NC, NS = _sc_env()
    e_per_t = E // NS
    G = e_per_t // CH
    mesh = plsc.VectorSubcoreMesh(core_axis_name="c", subcore_axis_name="s")
    zero = jnp.zeros((N, D), F32)

    @functools.partial(
        pl.kernel,
        mesh=mesh,
        out_type=jax.ShapeDtypeStruct((NC, N, D), F32),
        scratch_types=[
            pltpu.VMEM((CH,), jnp.int32),
            pltpu.VMEM((CH,), jnp.int32),
            pltpu.VMEM((CH, D), F32),
            pltpu.VMEM((CH, D), F32),
            pltpu.VMEM_SHARED((N, D), F32),
            pltpu.SemaphoreType.DMA,
            pltpu.SemaphoreType.DMA,
            pltpu.SemaphoreType.DMA,
            pltpu.SemaphoreType.DMA,
        ],
        compiler_params=pltpu.CompilerParams(use_tc_tiling_on_sc=True),
    )
    def sk(ra_hbm, rb_hbm, idx_hbm, zero_hbm, out_hbm,
           i0, i1, r0, r1, acc_sh, sr0, sr1, sa0, sa1):
        cid = lax.axis_index("c")
        sid = lax.axis_index("s")

        @pl.when(sid == 0)
        def _():
            pltpu.sync_copy(zero_hbm, acc_sh)

        plsc.subcore_barrier()
        base = sid * e_per_t
        T = G // 2

        def run(rows_hbm):
            def fetch(off, i_v, r_v, sr):
                pltpu.sync_copy(idx_hbm.at[pl.ds(off, CH)], i_v)
                pltpu.async_copy(rows_hbm.at[pl.ds(off, CH)], r_v, sr)

            def wait_fetch(r_v, sr):
                pltpu.make_async_copy(
                    rows_hbm.at[pl.ds(0, CH)], r_v, sr).wait()

            def wait_add(i_v, r_v, sa):
                pltpu.make_async_copy(r_v, acc_sh.at[i_v], sa).wait()

            def body(t, carry):
                off0 = base + (2 * t) * CH
                off1 = off0 + CH

                @pl.when(t > 0)
                def _():
                    wait_add(i0, r0, sa0)

                fetch(off0, i0, r0, sr0)

                @pl.when(t > 0)
                def _():
                    wait_add(i1, r1, sa1)

                fetch(off1, i1, r1, sr1)
                wait_fetch(r0, sr0)
                pltpu.async_copy(r0, acc_sh.at[i0], sa0, add=True)
                wait_fetch(r1, sr1)
                pltpu.async_copy(r1, acc_sh.at[i1], sa1, add=True)
                return carry

            lax.fori_loop(0, T, body, 0)
            if G % 2:
                offt = base + (G - 1) * CH
                wait_add(i0, r0, sa0)
                fetch(offt, i0, r0, sr0)
                wait_fetch(r0, sr0)
                pltpu.async_copy(r0, acc_sh.at[i0], sa0, add=True)
            wait_add(i0, r0, sa0)
            wait_add(i1, r1, sa1)

        @pl.when(cid == 0)
        def _():
            run(ra_hbm)

        @pl.when(cid == 1)
        def _():
            run(rb_hbm)

        plsc.subcore_barrier()

        @pl.when(sid == 0)
        def _():
            pltpu.sync_copy(acc_sh, out_hbm.at[cid])

    return sk(rows_a, rows_b, idx, zero)


def _edge_geom(vecT, rcij):
    E = rcij.shape[0]
    rcij3 = rcij.reshape(E // EBG, 1, EBG)

    def body(vt_ref, rc_ref, ep_ref):
        vx = vt_ref[0:1, :]
        vy = vt_ref[1:2, :]
        vz = vt_ref[2:3, :]
        d = jnp.sqrt(vx * vx + vy * vy + vz * vz)
        inv = 1.0 / d
        ux = vx * inv
        uy = vy * inv
        uz = vz * inv
        ones = jnp.ones((1, EBG), F32)
        Y = jnp.concatenate([
            0.28209479 * ones,
            0.48860251 * ux, 0.48860251 * uy, 0.48860251 * uz,
            1.09254843 * ux * uy, 1.09254843 * uy * uz,
            0.31539157 * (3.0 * uz * uz - 1.0),
            1.09254843 * ux * uz, 0.54627422 * (ux * ux - uy * uy),
        ], axis=0)
        swg = 0.5 * (1.0 + jnp.cos(np.float32(np.pi / CUTC) * d))
        swg = swg * (d < CUTC).astype(F32)
        rcij = rc_ref[0]
        tcl = jnp.clip((d - 0.5 * rcij) / (0.1 * rcij), 0.0, 1.0)
        sshort = 0.5 * (1.0 - jnp.cos(np.float32(np.pi) * tcl))
        switch = swg * sshort
        step = np.float32((CUTC - 0.8) / (RBC - 1))
        cen = np.float32(0.8) + step * lax.broadcasted_iota(
            jnp.int32, (RBC, 1), 0).astype(F32)
        eta = np.float32(1.0) / (step * step)
        dd = d - cen
        rt = jnp.exp(-eta * dd * dd) * sshort
        epT = jnp.concatenate(
            [switch, rt, Y, jnp.zeros((6, EBG), F32)], axis=0)
        ep_ref[...] = epT.T

    return pl.pallas_call(
        body,
        grid=(E // EBG,),
        in_specs=[
            pl.BlockSpec((3, EBG), lambda i: (0, i)),
            pl.BlockSpec((1, 1, EBG), lambda i: (i, 0, 0)),
        ],
        out_specs=pl.BlockSpec((EBG, EPREP_D), lambda i: (i, 0)),
        out_shape=jax.ShapeDtypeStruct((E, EPREP_D), F32),
    )(vecT, rcij3)


def _edge_attn(srows, drows, eprep, eoff, wp, bp2):
    E = srows.shape[0]
    epb = eoff // EB

    def body(s_ref, d_ref, ep_ref, wp_ref, bp_ref, me_ref, vie_ref):
        q = s_ref[...]
        dr = d_ref[...]
        ep = ep_ref[...]
        switch = ep[:, 0:1]
        rt = ep[:, 1:17]
        Y = ep[:, 17:26]
        k = dr[:, 0:128]
        v = dr[:, 128:256]
        tt = dr[:, 256:258]
        p = _swish(_dot(rt, wp_ref[...]) + bp_ref[...])
        qkp = q * k * p
        HA = HC * AC
        S = (lax.broadcasted_iota(jnp.int32, (HA, HC), 0) // AC
             == lax.broadcasted_iota(jnp.int32, (HA, HC), 1)).astype(F32)
        a = _dot(qkp, S) * (switch * np.float32(1.0 / np.sqrt(AC)))
        ST = (lax.broadcasted_iota(jnp.int32, (HC, HA), 0)
              == lax.broadcasted_iota(jnp.int32, (HC, HA), 1) // AC).astype(F32)
        me_ref[...] = _dot(a, ST) * v
        tco = tt * switch
        vie_ref[...] = jnp.concatenate(
            [tco[:, 0:1] * Y, tco[:, 1:2] * Y,
             jnp.zeros((EB, VIE_D - 18), F32)], axis=1)

    return pl.pallas_call(
        body,
        grid=(E // EB,),
        in_specs=[
            pl.BlockSpec((EB, DIMC), lambda i: (i, 0)),
            pl.BlockSpec((EB, DST_D), lambda i: (i, 0)),
            pl.BlockSpec((EB, EPREP_D), lambda i: (i + epb, 0)),
            pl.BlockSpec((RBC, DIMC), lambda i: (0, 0)),
            pl.BlockSpec((1, DIMC), lambda i: (0, 0)),
        ],
        out_specs=[
            pl.BlockSpec((EB, DIMC), lambda i: (i, 0)),
            pl.BlockSpec((EB, VIE_D), lambda i: (i, 0)),
        ],
        out_shape=[
            jax.ShapeDtypeStruct((E, DIMC), F32),
            jax.ShapeDtypeStruct((E, VIE_D), F32),
        ],
    )(srows, drows, eprep, wp, bp2)


def _tn_from_vi(vi):
    s = vi * vi
    return jnp.concatenate([
        s[:, 0:1], s[:, 9:10],
        jnp.sum(s[:, 1:4], axis=1, keepdims=True),
        jnp.sum(s[:, 10:13], axis=1, keepdims=True),
        jnp.sum(s[:, 4:9], axis=1, keepdims=True),
        jnp.sum(s[:, 13:18], axis=1, keepdims=True),
    ], axis=1)


def _node_update0(mpa, mpb, xi, wu_m, wu_t, bu2, Wo, Wq, Wk, Wv, Wt):
    N = xi.shape[0]

    def body(mpa_ref, mpb_ref, xi_ref,
             wum_ref, wut_ref, bu_ref, wo_ref,
             wq_ref, wk_ref, wv_ref, wt_ref,
             xo_ref, vio_ref, src_ref, dst_ref):
        mi = mpa_ref[0] + mpb_ref[0]
        vi = (mpa_ref[1] + mpb_ref[1])[:, 0:18]
        tn = _tn_from_vi(vi)
        h = _dot(mi, wum_ref[...]) + _dot(tn, wut_ref[...]) + bu_ref[...]
        upd = _dot(_swish(h), wo_ref[...])
        xn = _lnorm(xi_ref[...] + upd)
        xo_ref[...] = xn
        vio_ref[...] = vi
        src_ref[...] = _dot(xn, wq_ref[...])
        k = _dot(xn, wk_ref[...])
        v = _dot(xn, wv_ref[...])
        t = _dot(xn, wt_ref[...])
        dst_ref[...] = jnp.concatenate(
            [k, v, t, jnp.zeros((NB, DST_D - 258), F32)], axis=1)

    wspec = lambda r, c: pl.BlockSpec((r, c), lambda i: (0, 0))
    return pl.pallas_call(
        body,
        grid=(N // NB,),
        in_specs=[
            pl.BlockSpec((2, NB, DIMC), lambda i: (0, i, 0)),
            pl.BlockSpec((2, NB, DIMC), lambda i: (0, i, 0)),
            pl.BlockSpec((NB, DIMC), lambda i: (i, 0)),
            wspec(DIMC, DIMC), wspec(6, DIMC), wspec(1, DIMC),
            wspec(DIMC, DIMC), wspec(DIMC, DIMC), wspec(DIMC, DIMC),
            wspec(DIMC, DIMC), wspec(DIMC, TC_),
        ],
        out_specs=[
            pl.BlockSpec((NB, DIMC), lambda i: (i, 0)),
            pl.BlockSpec((NB, 18), lambda i: (i, 0)),
            pl.BlockSpec((NB, DIMC), lambda i: (i, 0)),
            pl.BlockSpec((NB, DST_D), lambda i: (i, 0)),
        ],
        out_shape=[
            jax.ShapeDtypeStruct((N, DIMC), F32),
            jax.ShapeDtypeStruct((N, 18), F32),
            jax.ShapeDtypeStruct((N, DIMC), F32),
            jax.ShapeDtypeStruct((N, DST_D), F32),
        ],
    )(mpa, mpb, xi, wu_m, wu_t, bu2, Wo, Wq, Wk, Wv, Wt)


def _node_update1(mpa, mpb, xi, vi_prev, wu_m, wu_t, bu2, Wo):
    N = xi.shape[0]

    def body(mpa_ref, mpb_ref, xi_ref, vip_ref,
             wum_ref, wut_ref, bu_ref, wo_ref, xo_ref):
        mi = mpa_ref[0] + mpb_ref[0]
        vi = vip_ref[...] + (mpa_ref[1] + mpb_ref[1])[:, 0:18]
        tn = _tn_from_vi(vi)
        h = _dot(mi, wum_ref[...]) + _dot(tn, wut_ref[...]) + bu_ref[...]
        upd = _dot(_swish(h), wo_ref[...])
        xo_ref[...] = _lnorm(xi_ref[...] + upd)

    wspec = lambda r, c: pl.BlockSpec((r, c), lambda i: (0, 0))
    return pl.pallas_call(
        body,
        grid=(N // NB,),
        in_specs=[
            pl.BlockSpec((2, NB, DIMC), lambda i: (0, i, 0)),
            pl.BlockSpec((2, NB, DIMC), lambda i: (0, i, 0)),
            pl.BlockSpec((NB, DIMC), lambda i: (i, 0)),
            pl.BlockSpec((NB, 18), lambda i: (i, 0)),
            wspec(DIMC, DIMC), wspec(6, DIMC), wspec(1, DIMC),
            wspec(DIMC, DIMC),
        ],
        out_specs=pl.BlockSpec((NB, DIMC), lambda i: (i, 0)),
        out_shape=jax.ShapeDtypeStruct((N, DIMC), F32),
    )(mpa, mpb, xi, vi_prev, wu_m, wu_t, bu2, Wo)


def kernel(species, edge_src, edge_dst, vec, params):
    N = species.shape[0]
    sp2 = species.reshape(N, 1).astype(jnp.int32)
    esrc = edge_src.astype(jnp.int32)
    edst = edge_dst.astype(jnp.int32)
    vecT = vec.astype(F32).T
    cov1 = jnp.asarray(_cov_np)
    lp0, lp1 = params["layers"]

    def split_wu(lp):
        return (lp["Wu"][:DIMC], lp["Wu"][DIMC:],
                lp["bu"].reshape(1, DIMC), lp["bp"].reshape(1, DIMC))

    wu_m0, wu_t0, bu0, bp0 = split_wu(lp0)
    wu_m1, wu_t1, bu1, bp1 = split_wu(lp1)

    xi0, src0, dst0 = _node_init(
        sp2, params["Zt"], params["Wsl"],
        lp0["Wq"], lp0["Wk"], lp0["Wv"], lp0["Wt"])

    rcij = _sc_rcij(sp2.reshape(N), cov1, esrc, edst)
    eprep = _edge_geom(vecT, rcij)

    E = esrc.shape[0]
    HE = E // 2
    halves = [(esrc[:HE], edst[:HE], 0), (esrc[HE:], edst[HE:], HE)]

    def layer(src_t, dst_t, wp, bp):
        rows = [_sc_gather2(src_t, dst_t, es, ed, ch=40)
                for (es, ed, _) in halves]
        me0, vie0 = _edge_attn(rows[0][0], rows[0][1], eprep, 0, wp, bp)
        mp0 = _sc_scatter_add2(me0, vie0, halves[0][0], N, ch=40)
        me1, vie1 = _edge_attn(rows[1][0], rows[1][1], eprep, HE, wp, bp)
        mp1 = _sc_scatter_add2(me1, vie1, halves[1][0], N, ch=40)
        return mp0, mp1

    mpa, mpb = layer(src0, dst0, lp0["Wp"], bp0)
    xi1, vi1, src1, dst1 = _node_update0(
        mpa, mpb, xi0, wu_m0, wu_t0, bu0, lp0["Wo"],
        lp1["Wq"], lp1["Wk"], lp1["Wv"], lp1["Wt"])

    mpa, mpb = layer(src1, dst1, lp1["Wp"], bp1)
    xi2 = _node_update1(mpa, mpb, xi1, vi1, wu_m1, wu_t1, bu1, lp1["Wo"])
    return xi2


# speedup vs baseline: 1.0012x; 1.0012x over previous
import functools

import numpy as np
import jax
import jax.numpy as jnp
from jax import lax
from jax.experimental import pallas as pl
from jax.experimental.pallas import tpu as pltpu
from jax.experimental.pallas import tpu_sc as plsc

F32 = jnp.float32
_HI = lax.Precision.HIGHEST

DIMC = 128
AC = 16
HC = 8
TC_ = 2
NSPC = 64
RBC = 16
CUTC = 5.0

_cov_np = np.linspace(0.3, 1.6, NSPC).astype(np.float32)

NB = 1000
EB = 4000
EBG = 2560
CH = 80

DST_D = 384
VIE_D = 128
EPREP_D = 32


def _swish(x):
    return x / (1.0 + jnp.exp(-x))


def _lnorm(x):
    mu = jnp.mean(x, axis=-1, keepdims=True)
    dx = x - mu
    var = jnp.mean(dx * dx, axis=-1, keepdims=True)
    return dx * lax.rsqrt(1e-06 + var)


def _dot(a, b):
    return jnp.dot(a, b, precision=_HI, preferred_element_type=F32)


def _sc_env():
    info = plsc.get_sparse_core_info()
    return info.num_cores, info.num_subcores


def _node_init(sp2, Zt, Wsl, Wq, Wk, Wv, Wt):
    N = sp2.shape[0]

    def body(sp_ref, zt_ref, wsl_ref, wq_ref, wk_ref, wv_ref, wt_ref,
             xi_ref, src_ref, dst_ref):
        sp = sp_ref[...]
        ids = lax.broadcasted_iota(jnp.int32, (NB, NSPC), 1)
        oh = (ids == sp).astype(F32)
        z = _dot(oh, zt_ref[...])
        xi = _lnorm(_dot(z, wsl_ref[...]))
        q = _dot(xi, wq_ref[...])
        k = _dot(xi, wk_ref[...])
        v = _dot(xi, wv_ref[...])
        t = _dot(xi, wt_ref[...])
        xi_ref[...] = xi
        src_ref[...] = q
        dst_ref[...] = jnp.concatenate(
            [k, v, t, jnp.zeros((NB, DST_D - 258), F32)], axis=1)

    wspec = lambda r, c: pl.BlockSpec((r, c), lambda i: (0, 0))
    return pl.pallas_call(
        body,
        grid=(N // NB,),
        in_specs=[
            pl.BlockSpec((NB, 1), lambda i: (i, 0)),
            wspec(NSPC, 16), wspec(16, DIMC),
            wspec(DIMC, DIMC), wspec(DIMC, DIMC), wspec(DIMC, DIMC),
            wspec(DIMC, TC_),
        ],
        out_specs=[
            pl.BlockSpec((NB, DIMC), lambda i: (i, 0)),
            pl.BlockSpec((NB, DIMC), lambda i: (i, 0)),
            pl.BlockSpec((NB, DST_D), lambda i: (i, 0)),
        ],
        out_shape=[
            jax.ShapeDtypeStruct((N, DIMC), F32),
            jax.ShapeDtypeStruct((N, DIMC), F32),
            jax.ShapeDtypeStruct((N, DST_D), F32),
        ],
    )(sp2, Zt, Wsl, Wq, Wk, Wv, Wt)


def _sc_rcij(species, cov1d, esrc, edst):
    N = species.shape[0]
    E = esrc.shape[0]
    NC, NS = _sc_env()
    e_per_w = E // (NC * NS)
    G = e_per_w // CH
    mesh = plsc.VectorSubcoreMesh(core_axis_name="c", subcore_axis_name="s")

    @functools.partial(
        pl.kernel,
        mesh=mesh,
        out_type=jax.ShapeDtypeStruct((E,), F32),
        scratch_types=[
            pltpu.VMEM((N,), jnp.int32),
            pltpu.VMEM((NSPC,), F32),
            pltpu.VMEM((CH,), jnp.int32),
            pltpu.VMEM((CH,), jnp.int32),
            pltpu.VMEM((CH,), F32),
        ],
        compiler_params=pltpu.CompilerParams(needs_layout_passes=False),
    )
    def rk(sp_hbm, cov_hbm, es_hbm, ed_hbm, out_hbm, sp_v, cov_v, is_v, id_v,
           o_v):
        wid = lax.axis_index("s") * NC + lax.axis_index("c")
        base = wid * e_per_w
        pltpu.sync_copy(sp_hbm, sp_v)
        pltpu.sync_copy(cov_hbm, cov_v)

        def step(g, carry):
            off = base + g * CH
            pltpu.sync_copy(es_hbm.at[pl.ds(off, CH)], is_v)
            pltpu.sync_copy(ed_hbm.at[pl.ds(off, CH)], id_v)
            for j in range(CH // 16):
                si = is_v[pl.ds(16 * j, 16)]
                di = id_v[pl.ds(16 * j, 16)]
                sps = plsc.load_gather(sp_v, [si])
                spd = plsc.load_gather(sp_v, [di])
                vs = plsc.load_gather(cov_v, [sps])
                vd = plsc.load_gather(cov_v, [spd])
                o_v[pl.ds(16 * j, 16)] = vs + vd
            pltpu.sync_copy(o_v, out_hbm.at[pl.ds(off, CH)])
            return carry

        lax.fori_loop(0, G, step, 0)

    return rk(species, cov1d, esrc, edst)


def _sc_gather2(tbl_s, tbl_d, esrc, edst, ch=CH):
    CH = ch
    N, DS = tbl_s.shape
    DD = tbl_d.shape[1]
    E = esrc.shape[0]
    NC, NS = _sc_env()
    e_per_w = E // (NC * NS)
    G = e_per_w // CH
    mesh = plsc.VectorSubcoreMesh(core_axis_name="c", subcore_axis_name="s")

    @functools.partial(
        pl.kernel,
        mesh=mesh,
        out_type=[
            jax.ShapeDtypeStruct((E, DS), F32),
            jax.ShapeDtypeStruct((E, DD), F32),
        ],
        scratch_types=[
            pltpu.VMEM((CH,), jnp.int32),
            pltpu.VMEM((CH,), jnp.int32),
            pltpu.VMEM((CH,), jnp.int32),
            pltpu.VMEM((CH,), jnp.int32),
            pltpu.VMEM((CH, DS), F32),
            pltpu.VMEM((CH, DD), F32),
            pltpu.VMEM((CH, DS), F32),
            pltpu.VMEM((CH, DD), F32),
            pltpu.SemaphoreType.DMA,
            pltpu.SemaphoreType.DMA,
            pltpu.SemaphoreType.DMA,
            pltpu.SemaphoreType.DMA,
        ],
        compiler_params=pltpu.CompilerParams(use_tc_tiling_on_sc=True),
    )
    def gk(ts_hbm, td_hbm, es_hbm, ed_hbm, os_hbm, od_hbm,
           is0, id0, is1, id1, rs0, rd0, rs1, rd1, sg0, sg1, so0, so1):
        wid = lax.axis_index("s") * NC + lax.axis_index("c")
        base = wid * e_per_w
        T = G // 2

        def fetch(off, is_v, id_v, rs_v, rd_v, sg):
            pltpu.sync_copy(es_hbm.at[pl.ds(off, CH)], is_v)
            pltpu.sync_copy(ed_hbm.at[pl.ds(off, CH)], id_v)
            pltpu.async_copy(ts_hbm.at[is_v], rs_v, sg)
            pltpu.async_copy(td_hbm.at[id_v], rd_v, sg)

        def wait_gather(is_v, id_v, rs_v, rd_v, sg):
            pltpu.make_async_copy(ts_hbm.at[is_v], rs_v, sg).wait()
            pltpu.make_async_copy(td_hbm.at[id_v], rd_v, sg).wait()

        def put(off, rs_v, rd_v, so):
            pltpu.async_copy(rs_v, os_hbm.at[pl.ds(off, CH)], so)
            pltpu.async_copy(rd_v, od_hbm.at[pl.ds(off, CH)], so)

        def wait_put(rs_v, rd_v, so):
            pltpu.make_async_copy(rs_v, os_hbm.at[pl.ds(0, CH)], so).wait()
            pltpu.make_async_copy(rd_v, od_hbm.at[pl.ds(0, CH)], so).wait()

        def body(t, carry):
            off0 = base + (2 * t) * CH
            off1 = off0 + CH

            @pl.when(t > 0)
            def _():
                wait_put(rs0, rd0, so0)

            fetch(off0, is0, id0, rs0, rd0, sg0)

            @pl.when(t > 0)
            def _():
                wait_put(rs1, rd1, so1)

            fetch(off1, is1, id1, rs1, rd1, sg1)
            wait_gather(is0, id0, rs0, rd0, sg0)
            put(off0, rs0, rd0, so0)
            wait_gather(is1, id1, rs1, rd1, sg1)
            put(off1, rs1, rd1, so1)
            return carry

        lax.fori_loop(0, T, body, 0)
        if G % 2:
            offt = base + (G - 1) * CH
            wait_put(rs0, rd0, so0)
            fetch(offt, is0, id0, rs0, rd0, sg0)
            wait_gather(is0, id0, rs0, rd0, sg0)
            put(offt, rs0, rd0, so0)
        wait_put(rs0, rd0, so0)
        wait_put(rs1, rd1, so1)

    return gk(tbl_s, tbl_d, esrc, edst)


def _sc_scatter_add2(rows_a, rows_b, idx, N, ch=CH):
    CH = ch
    E, D = rows_a.shape
    NC, NS = _sc_env()
    e_per_t = E // NS
    G = e_per_t // CH
    mesh = plsc.VectorSubcoreMesh(core_axis_name="c", subcore_axis_name="s")
    zero = jnp.zeros((N, D), F32)

    @functools.partial(
        pl.kernel,
        mesh=mesh,
        out_type=jax.ShapeDtypeStruct((NC, N, D), F32),
        scratch_types=[
            pltpu.VMEM((CH,), jnp.int32),
            pltpu.VMEM((CH,), jnp.int32),
            pltpu.VMEM((CH, D), F32),
            pltpu.VMEM((CH, D), F32),
            pltpu.VMEM_SHARED((N, D), F32),
            pltpu.SemaphoreType.DMA,
            pltpu.SemaphoreType.DMA,
            pltpu.SemaphoreType.DMA,
            pltpu.SemaphoreType.DMA,
        ],
        compiler_params=pltpu.CompilerParams(use_tc_tiling_on_sc=True),
    )
    def sk(ra_hbm, rb_hbm, idx_hbm, zero_hbm, out_hbm,
           i0, i1, r0, r1, acc_sh, sr0, sr1, sa0, sa1):
        cid = lax.axis_index("c")
        sid = lax.axis_index("s")

        @pl.when(sid == 0)
        def _():
            pltpu.sync_copy(zero_hbm, acc_sh)

        plsc.subcore_barrier()
        base = sid * e_per_t
        T = G // 2

        def run(rows_hbm):
            def fetch(off, i_v, r_v, sr):
                pltpu.sync_copy(idx_hbm.at[pl.ds(off, CH)], i_v)
                pltpu.async_copy(rows_hbm.at[pl.ds(off, CH)], r_v, sr)

            def wait_fetch(r_v, sr):
                pltpu.make_async_copy(
                    rows_hbm.at[pl.ds(0, CH)], r_v, sr).wait()

            def wait_add(i_v, r_v, sa):
                pltpu.make_async_copy(r_v, acc_sh.at[i_v], sa).wait()

            def body(t, carry):
                off0 = base + (2 * t) * CH
                off1 = off0 + CH

                @pl.when(t > 0)
                def _():
                    wait_add(i0, r0, sa0)

                fetch(off0, i0, r0, sr0)

                @pl.when(t > 0)
                def _():
                    wait_add(i1, r1, sa1)

                fetch(off1, i1, r1, sr1)
                wait_fetch(r0, sr0)
                pltpu.async_copy(r0, acc_sh.at[i0], sa0, add=True)
                wait_fetch(r1, sr1)
                pltpu.async_copy(r1, acc_sh.at[i1], sa1, add=True)
                return carry

            lax.fori_loop(0, T, body, 0)
            if G % 2:
                offt = base + (G - 1) * CH
                wait_add(i0, r0, sa0)
                fetch(offt, i0, r0, sr0)
                wait_fetch(r0, sr0)
                pltpu.async_copy(r0, acc_sh.at[i0], sa0, add=True)
            wait_add(i0, r0, sa0)
            wait_add(i1, r1, sa1)

        @pl.when(cid == 0)
        def _():
            run(ra_hbm)

        @pl.when(cid == 1)
        def _():
            run(rb_hbm)

        plsc.subcore_barrier()

        @pl.when(sid == 0)
        def _():
            pltpu.sync_copy(acc_sh, out_hbm.at[cid])

    return sk(rows_a, rows_b, idx, zero)


def _edge_geom(vecT, rcij):
    E = rcij.shape[0]
    rcij3 = rcij.reshape(E // EBG, 1, EBG)

    def body(vt_ref, rc_ref, ep_ref):
        vx = vt_ref[0:1, :]
        vy = vt_ref[1:2, :]
        vz = vt_ref[2:3, :]
        d = jnp.sqrt(vx * vx + vy * vy + vz * vz)
        inv = 1.0 / d
        ux = vx * inv
        uy = vy * inv
        uz = vz * inv
        ones = jnp.ones((1, EBG), F32)
        Y = jnp.concatenate([
            0.28209479 * ones,
            0.48860251 * ux, 0.48860251 * uy, 0.48860251 * uz,
            1.09254843 * ux * uy, 1.09254843 * uy * uz,
            0.31539157 * (3.0 * uz * uz - 1.0),
            1.09254843 * ux * uz, 0.54627422 * (ux * ux - uy * uy),
        ], axis=0)
        swg = 0.5 * (1.0 + jnp.cos(np.float32(np.pi / CUTC) * d))
        swg = swg * (d < CUTC).astype(F32)
        rcij = rc_ref[0]
        tcl = jnp.clip((d - 0.5 * rcij) / (0.1 * rcij), 0.0, 1.0)
        sshort = 0.5 * (1.0 - jnp.cos(np.float32(np.pi) * tcl))
        switch = swg * sshort
        step = np.float32((CUTC - 0.8) / (RBC - 1))
        cen = np.float32(0.8) + step * lax.broadcasted_iota(
            jnp.int32, (RBC, 1), 0).astype(F32)
        eta = np.float32(1.0) / (step * step)
        dd = d - cen
        rt = jnp.exp(-eta * dd * dd) * sshort
        epT = jnp.concatenate(
            [switch, rt, Y, jnp.zeros((6, EBG), F32)], axis=0)
        ep_ref[...] = epT.T

    return pl.pallas_call(
        body,
        grid=(E // EBG,),
        in_specs=[
            pl.BlockSpec((3, EBG), lambda i: (0, i)),
            pl.BlockSpec((1, 1, EBG), lambda i: (i, 0, 0)),
        ],
        out_specs=pl.BlockSpec((EBG, EPREP_D), lambda i: (i, 0)),
        out_shape=jax.ShapeDtypeStruct((E, EPREP_D), F32),
    )(vecT, rcij3)


def _edge_attn(srows, drows, eprep, eoff, wp, bp2):
    E = srows.shape[0]
    epb = eoff // EB

    def body(s_ref, d_ref, ep_ref, wp_ref, bp_ref, me_ref, vie_ref):
        q = s_ref[...]
        dr = d_ref[...]
        ep = ep_ref[...]
        switch = ep[:, 0:1]
        rt = ep[:, 1:17]
        Y = ep[:, 17:26]
        k = dr[:, 0:128]
        v = dr[:, 128:256]
        tt = dr[:, 256:258]
        p = _swish(_dot(rt, wp_ref[...]) + bp_ref[...])
        qkp = q * k * p
        HA = HC * AC
        S = (lax.broadcasted_iota(jnp.int32, (HA, HC), 0) // AC
             == lax.broadcasted_iota(jnp.int32, (HA, HC), 1)).astype(F32)
        a = _dot(qkp, S) * (switch * np.float32(1.0 / np.sqrt(AC)))
        ST = (lax.broadcasted_iota(jnp.int32, (HC, HA), 0)
              == lax.broadcasted_iota(jnp.int32, (HC, HA), 1) // AC).astype(F32)
        me_ref[...] = _dot(a, ST) * v
        tco = tt * switch
        vie_ref[...] = jnp.concatenate(
            [tco[:, 0:1] * Y, tco[:, 1:2] * Y,
             jnp.zeros((EB, VIE_D - 18), F32)], axis=1)

    return pl.pallas_call(
        body,
        grid=(E // EB,),
        in_specs=[
            pl.BlockSpec((EB, DIMC), lambda i: (i, 0)),
            pl.BlockSpec((EB, DST_D), lambda i: (i, 0)),
            pl.BlockSpec((EB, EPREP_D), lambda i: (i + epb, 0)),
            pl.BlockSpec((RBC, DIMC), lambda i: (0, 0)),
            pl.BlockSpec((1, DIMC), lambda i: (0, 0)),
        ],
        out_specs=[
            pl.BlockSpec((EB, DIMC), lambda i: (i, 0)),
            pl.BlockSpec((EB, VIE_D), lambda i: (i, 0)),
        ],
        out_shape=[
            jax.ShapeDtypeStruct((E, DIMC), F32),
            jax.ShapeDtypeStruct((E, VIE_D), F32),
        ],
    )(srows, drows, eprep, wp, bp2)


def _tn_from_vi(vi):
    s = vi * vi
    return jnp.concatenate([
        s[:, 0:1], s[:, 9:10],
        jnp.sum(s[:, 1:4], axis=1, keepdims=True),
        jnp.sum(s[:, 10:13], axis=1, keepdims=True),
        jnp.sum(s[:, 4:9], axis=1, keepdims=True),
        jnp.sum(s[:, 13:18], axis=1, keepdims=True),
    ], axis=1)


def _node_update0(mpa, mpb, xi, wu_m, wu_t, bu2, Wo, Wq, Wk, Wv, Wt):
    N = xi.shape[0]

    def body(mpa_ref, mpb_ref, xi_ref,
             wum_ref, wut_ref, bu_ref, wo_ref,
             wq_ref, wk_ref, wv_ref, wt_ref,
             xo_ref, vio_ref, src_ref, dst_ref):
        mi = mpa_ref[0] + mpb_ref[0]
        vi = (mpa_ref[1] + mpb_ref[1])[:, 0:18]
        tn = _tn_from_vi(vi)
        h = _dot(mi, wum_ref[...]) + _dot(tn, wut_ref[...]) + bu_ref[...]
        upd = _dot(_swish(h), wo_ref[...])
        xn = _lnorm(xi_ref[...] + upd)
        xo_ref[...] = xn
        vio_ref[...] = vi
        src_ref[...] = _dot(xn, wq_ref[...])
        k = _dot(xn, wk_ref[...])
        v = _dot(xn, wv_ref[...])
        t = _dot(xn, wt_ref[...])
        dst_ref[...] = jnp.concatenate(
            [k, v, t, jnp.zeros((NB, DST_D - 258), F32)], axis=1)

    wspec = lambda r, c: pl.BlockSpec((r, c), lambda i: (0, 0))
    return pl.pallas_call(
        body,
        grid=(N // NB,),
        in_specs=[
            pl.BlockSpec((2, NB, DIMC), lambda i: (0, i, 0)),
            pl.BlockSpec((2, NB, DIMC), lambda i: (0, i, 0)),
            pl.BlockSpec((NB, DIMC), lambda i: (i, 0)),
            wspec(DIMC, DIMC), wspec(6, DIMC), wspec(1, DIMC),
            wspec(DIMC, DIMC), wspec(DIMC, DIMC), wspec(DIMC, DIMC),
            wspec(DIMC, DIMC), wspec(DIMC, TC_),
        ],
        out_specs=[
            pl.BlockSpec((NB, DIMC), lambda i: (i, 0)),
            pl.BlockSpec((NB, 18), lambda i: (i, 0)),
            pl.BlockSpec((NB, DIMC), lambda i: (i, 0)),
            pl.BlockSpec((NB, DST_D), lambda i: (i, 0)),
        ],
        out_shape=[
            jax.ShapeDtypeStruct((N, DIMC), F32),
            jax.ShapeDtypeStruct((N, 18), F32),
            jax.ShapeDtypeStruct((N, DIMC), F32),
            jax.ShapeDtypeStruct((N, DST_D), F32),
        ],
    )(mpa, mpb, xi, wu_m, wu_t, bu2, Wo, Wq, Wk, Wv, Wt)


def _node_update1(mpa, mpb, xi, vi_prev, wu_m, wu_t, bu2, Wo):
    N = xi.shape[0]

    def body(mpa_ref, mpb_ref, xi_ref, vip_ref,
             wum_ref, wut_ref, bu_ref, wo_ref, xo_ref):
        mi = mpa_ref[0] + mpb_ref[0]
        vi = vip_ref[...] + (mpa_ref[1] + mpb_ref[1])[:, 0:18]
        tn = _tn_from_vi(vi)
        h = _dot(mi, wum_ref[...]) + _dot(tn, wut_ref[...]) + bu_ref[...]
        upd = _dot(_swish(h), wo_ref[...])
        xo_ref[...] = _lnorm(xi_ref[...] + upd)

    wspec = lambda r, c: pl.BlockSpec((r, c), lambda i: (0, 0))
    return pl.pallas_call(
        body,
        grid=(N // NB,),
        in_specs=[
            pl.BlockSpec((2, NB, DIMC), lambda i: (0, i, 0)),
            pl.BlockSpec((2, NB, DIMC), lambda i: (0, i, 0)),
            pl.BlockSpec((NB, DIMC), lambda i: (i, 0)),
            pl.BlockSpec((NB, 18), lambda i: (i, 0)),
            wspec(DIMC, DIMC), wspec(6, DIMC), wspec(1, DIMC),
            wspec(DIMC, DIMC),
        ],
        out_specs=pl.BlockSpec((NB, DIMC), lambda i: (i, 0)),
        out_shape=jax.ShapeDtypeStruct((N, DIMC), F32),
    )(mpa, mpb, xi, vi_prev, wu_m, wu_t, bu2, Wo)


def kernel(species, edge_src, edge_dst, vec, params):
    N = species.shape[0]
    sp2 = species.reshape(N, 1).astype(jnp.int32)
    esrc = edge_src.astype(jnp.int32)
    edst = edge_dst.astype(jnp.int32)
    vecT = vec.astype(F32).T
    cov1 = jnp.asarray(_cov_np)
    lp0, lp1 = params["layers"]

    def split_wu(lp):
        return (lp["Wu"][:DIMC], lp["Wu"][DIMC:],
                lp["bu"].reshape(1, DIMC), lp["bp"].reshape(1, DIMC))

    wu_m0, wu_t0, bu0, bp0 = split_wu(lp0)
    wu_m1, wu_t1, bu1, bp1 = split_wu(lp1)

    xi0, src0, dst0 = _node_init(
        sp2, params["Zt"], params["Wsl"],
        lp0["Wq"], lp0["Wk"], lp0["Wv"], lp0["Wt"])

    rcij = _sc_rcij(sp2.reshape(N), cov1, esrc, edst)
    eprep = _edge_geom(vecT, rcij)

    E = esrc.shape[0]
    HE = E // 2
    halves = [(esrc[:HE], edst[:HE], 0), (esrc[HE:], edst[HE:], HE)]

    def layer(src_t, dst_t, wp, bp):
        rows = [_sc_gather2(src_t, dst_t, es, ed, ch=40)
                for (es, ed, _) in halves]
        me0, vie0 = _edge_attn(rows[0][0], rows[0][1], eprep, 0, wp, bp)
        mp0 = _sc_scatter_add2(me0, vie0, halves[0][0], N, ch=40)
        me1, vie1 = _edge_attn(rows[1][0], rows[1][1], eprep, HE, wp, bp)
        mp1 = _sc_scatter_add2(me1, vie1, halves[1][0], N, ch=40)
        return mp0, mp1

    mpa, mpb = layer(src0, dst0, lp0["Wp"], bp0)
    xi1, vi1, src1, dst1 = _node_update0(
        mpa, mpb, xi0, wu_m0, wu_t0, bu0, lp0["Wo"],
        lp1["Wq"], lp1["Wk"], lp1["Wv"], lp1["Wt"])

    mpa, mpb = layer(src1, dst1, lp1["Wp"], bp1)
    xi2 = _node_update1(mpa, mpb, xi1, vi1, wu_m1, wu_t1, bu1, lp1["Wo"])
    return xi2
